# Initial kernel scaffold; baseline (speedup 1.0000x reference)
import functools

import jax
import jax.numpy as jnp
from jax import lax
from jax.experimental import pallas as pl
from jax.experimental.pallas import tpu as pltpu
from jax.experimental.pallas import tpu_sc as plsc

B = 16384
F = 26
L = 50
V = 100000
D = 32

NC = 2
NS = 16
NW = NC * NS
SPW = B // NW
S = 16
NCHUNK = SPW // S
ROWS = S * (F + 1)
SPID = S * F
SEQN = S * L

_mesh = plsc.VectorSubcoreMesh(core_axis_name="c", subcore_axis_name="s")


@functools.partial(
    pl.kernel,
    out_type=jax.ShapeDtypeStruct((B * (F + 1), D), jnp.float32),
    mesh=_mesh,
    scratch_types=[
        pltpu.VMEM((SPID,), jnp.int32),
        pltpu.VMEM((ROWS,), jnp.int32),
        pltpu.VMEM((SEQN,), jnp.int32),
        pltpu.VMEM((ROWS, D), jnp.float32),
        pltpu.VMEM((SEQN, D), jnp.float32),
        pltpu.VMEM((1, D), jnp.float32),
        pltpu.SemaphoreType.DMA,
    ],
)
def _sc_embed(sp_ids, sq_ids, tables, seq_tab, out,
              ids_v, idx_v, sqid_v, rows_v, seq_v, row0_v, sem):
    wid = lax.axis_index("s") * NC + lax.axis_index("c")
    pltpu.sync_copy(seq_tab.at[pl.ds(0, 1)], row0_v)
    lanes = lax.iota(jnp.int32, 16)

    def chunk_body(c, carry):
        s_base = wid * SPW + c * S
        pltpu.sync_copy(sp_ids.at[pl.ds(s_base * F, SPID)], ids_v)
        pltpu.sync_copy(sq_ids.at[pl.ds(s_base * L, SEQN)], sqid_v)

        for j in range(ROWS // 16):
            pos = lanes + (j * 16)
            f = pos % (F + 1)
            smp = pos // (F + 1)
            src = jnp.minimum(smp * F + f, SPID - 1)
            idv = plsc.load_gather(ids_v, [src])
            idx_v[pl.ds(j * 16, 16)] = jnp.where(f < F, idv + f * V, 0)

        copies = []
        for off in range(0, ROWS, 128):
            n = min(128, ROWS - off)
            copies.append(pltpu.async_copy(
                tables.at[idx_v.at[pl.ds(off, n)]], rows_v.at[pl.ds(off, n)], sem))
        for off in range(0, SEQN, 128):
            n = min(128, SEQN - off)
            copies.append(pltpu.async_copy(
                seq_tab.at[sqid_v.at[pl.ds(off, n)]], seq_v.at[pl.ds(off, n)], sem))
        for cp in copies:
            cp.wait()

        r0a = row0_v[0, pl.ds(0, 16)]
        r0b = row0_v[0, pl.ds(16, 16)]

        def sample_body(s, carry2):
            ob = s * L
            i0 = sqid_v[pl.ds(ob, 16)]
            i1 = sqid_v[pl.ds(ob + 16, 16)]
            i2 = sqid_v[pl.ds(ob + 32, 16)]
            i3 = sqid_v[pl.ds(ob + 34, 16)]
            cnt = (plsc.all_reduce_population_count(i0 != 0)
                   + plsc.all_reduce_population_count(i1 != 0)
                   + plsc.all_reduce_population_count(i2 != 0)
                   + plsc.all_reduce_population_count(
                       jnp.logical_and(i3 != 0, lanes >= 14)))
            acc0 = jnp.zeros((16,), jnp.float32)
            acc1 = jnp.zeros((16,), jnp.float32)
            for l in range(L):
                acc0 = acc0 + seq_v[ob + l, pl.ds(0, 16)]
                acc1 = acc1 + seq_v[ob + l, pl.ds(16, 16)]
            cf = cnt.astype(jnp.float32)
            nz = 50.0 - cf
            denom = jnp.maximum(cf, 1.0)
            p0 = jnp.where(cnt > 0, (acc0 - nz * r0a) / denom, 0.0)
            p1 = jnp.where(cnt > 0, (acc1 - nz * r0b) / denom, 0.0)
            orow = s * (F + 1) + F
            rows_v[orow, pl.ds(0, 16)] = p0
            rows_v[orow, pl.ds(16, 16)] = p1
            return carry2

        lax.fori_loop(0, S, sample_body, 0)
        pltpu.sync_copy(rows_v, out.at[pl.ds(s_base * (F + 1), ROWS)])
        return carry

    lax.fori_loop(0, NCHUNK, chunk_body, 0)


def kernel(sparse_ids, seq_ids, sparse_tables, seq_table):
    sp = sparse_ids.astype(jnp.int32).reshape(B * F)
    sq = seq_ids.astype(jnp.int32).reshape(B * L)
    tabs = sparse_tables.reshape(F * V, D)
    out = _sc_embed(sp, sq, tabs, seq_table)
    return out.reshape(B, F + 1, D)


# speedup vs baseline: 2.4085x; 2.4085x over previous
import functools

import jax
import jax.numpy as jnp
from jax import lax
from jax.experimental import pallas as pl
from jax.experimental.pallas import tpu as pltpu
from jax.experimental.pallas import tpu_sc as plsc

B = 16384
F = 26
L = 50
V = 100000
D = 32

NC = 2
NS = 16
NW = NC * NS
SPW = B // NW
S = 16
NCHUNK = SPW // S
ROWS = S * (F + 1)
SPID = S * F
SEQN = S * L

_mesh = plsc.VectorSubcoreMesh(core_axis_name="c", subcore_axis_name="s")


@functools.partial(
    pl.kernel,
    out_type=jax.ShapeDtypeStruct((B * (F + 1), D), jnp.float32),
    mesh=_mesh,
    compiler_params=pltpu.CompilerParams(
        needs_layout_passes=False, use_tc_tiling_on_sc=False),
    scratch_types=[
        pltpu.VMEM((SPID,), jnp.int32),
        pltpu.VMEM((ROWS,), jnp.int32),
        pltpu.VMEM((SEQN,), jnp.int32),
        pltpu.VMEM((ROWS, D), jnp.float32),
        pltpu.VMEM((SEQN, D), jnp.float32),
        pltpu.VMEM((1, D), jnp.float32),
        pltpu.SemaphoreType.DMA,
    ],
)
def _sc_embed(sp_ids, sq_ids, tables, seq_tab, out,
              ids_v, idx_v, sqid_v, rows_v, seq_v, row0_v, sem):
    wid = lax.axis_index("s") * NC + lax.axis_index("c")
    pltpu.sync_copy(seq_tab.at[pl.ds(0, 1)], row0_v)
    lanes = lax.iota(jnp.int32, 16)

    def chunk_body(c, carry):
        s_base = wid * SPW + c * S
        pltpu.sync_copy(sp_ids.at[pl.ds(s_base * F, SPID)], ids_v)
        pltpu.sync_copy(sq_ids.at[pl.ds(s_base * L, SEQN)], sqid_v)

        for j in range(ROWS // 16):
            pos = lanes + (j * 16)
            f = pos % (F + 1)
            smp = pos // (F + 1)
            src = jnp.minimum(smp * F + f, SPID - 1)
            idv = plsc.load_gather(ids_v, [src])
            idx_v[pl.ds(j * 16, 16)] = jnp.where(f < F, idv + f * V, 0)

        copies = []
        for off in range(0, ROWS, 128):
            n = min(128, ROWS - off)
            copies.append(pltpu.async_copy(
                tables.at[idx_v.at[pl.ds(off, n)]], rows_v.at[pl.ds(off, n)], sem))
        for off in range(0, SEQN, 128):
            n = min(128, SEQN - off)
            copies.append(pltpu.async_copy(
                seq_tab.at[sqid_v.at[pl.ds(off, n)]], seq_v.at[pl.ds(off, n)], sem))
        for cp in copies:
            cp.wait()

        r0a = row0_v[0, pl.ds(0, 16)]
        r0b = row0_v[0, pl.ds(16, 16)]

        def sample_body(s, carry2):
            ob = s * L
            i0 = sqid_v[pl.ds(ob, 16)]
            i1 = sqid_v[pl.ds(ob + 16, 16)]
            i2 = sqid_v[pl.ds(ob + 32, 16)]
            i3 = sqid_v[pl.ds(ob + 34, 16)]
            cnt = (plsc.all_reduce_population_count(i0 != 0)
                   + plsc.all_reduce_population_count(i1 != 0)
                   + plsc.all_reduce_population_count(i2 != 0)
                   + plsc.all_reduce_population_count(
                       jnp.logical_and(i3 != 0, lanes >= 14)))
            acc0 = jnp.zeros((16,), jnp.float32)
            acc1 = jnp.zeros((16,), jnp.float32)
            for l in range(L):
                acc0 = acc0 + seq_v[ob + l, pl.ds(0, 16)]
                acc1 = acc1 + seq_v[ob + l, pl.ds(16, 16)]
            cf = cnt.astype(jnp.float32)
            nz = 50.0 - cf
            denom = jnp.maximum(cf, 1.0)
            p0 = jnp.where(cnt > 0, (acc0 - nz * r0a) / denom, 0.0)
            p1 = jnp.where(cnt > 0, (acc1 - nz * r0b) / denom, 0.0)
            orow = s * (F + 1) + F
            rows_v[orow, pl.ds(0, 16)] = p0
            rows_v[orow, pl.ds(16, 16)] = p1
            return carry2

        lax.fori_loop(0, S, sample_body, 0)
        pltpu.sync_copy(rows_v, out.at[pl.ds(s_base * (F + 1), ROWS)])
        return carry

    lax.fori_loop(0, NCHUNK, chunk_body, 0)


def kernel(sparse_ids, seq_ids, sparse_tables, seq_table):
    sp = sparse_ids.astype(jnp.int32).reshape(B * F)
    sq = seq_ids.astype(jnp.int32).reshape(B * L)
    tabs = sparse_tables.reshape(F * V, D)
    out = _sc_embed(sp, sq, tabs, seq_table)
    return out.reshape(B, F + 1, D)


# speedup vs baseline: 2.4107x; 1.0009x over previous
import functools

import jax
import jax.numpy as jnp
from jax import lax
from jax.experimental import pallas as pl
from jax.experimental.pallas import tpu as pltpu
from jax.experimental.pallas import tpu_sc as plsc

B = 16384
F = 26
L = 50
V = 100000
D = 32

NC = 2
NS = 16
NW = NC * NS
SPW = B // NW
S = 16
NCHUNK = SPW // S
ROWS = S * (F + 1)
SPID = S * F
SEQN = S * L
GSUB = 128

_mesh = plsc.VectorSubcoreMesh(core_axis_name="c", subcore_axis_name="s")

_buf_set = [
    pltpu.VMEM((SPID,), jnp.int32),
    pltpu.VMEM((ROWS,), jnp.int32),
    pltpu.VMEM((SEQN,), jnp.int32),
    pltpu.VMEM((ROWS, D), jnp.float32),
    pltpu.VMEM((SEQN, D), jnp.float32),
    pltpu.SemaphoreType.DMA,
    pltpu.SemaphoreType.DMA,
]


@functools.partial(
    pl.kernel,
    out_type=jax.ShapeDtypeStruct((B * (F + 1), D), jnp.float32),
    mesh=_mesh,
    compiler_params=pltpu.CompilerParams(
        needs_layout_passes=False, use_tc_tiling_on_sc=False),
    scratch_types=_buf_set + _buf_set + [
        pltpu.VMEM((ROWS,), jnp.int32),
        pltpu.VMEM((ROWS,), jnp.int32),
        pltpu.VMEM((1, D), jnp.float32),
    ],
)
def _sc_embed(sp_ids, sq_ids, tables, seq_tab, out, *scr):
    b0, b1 = scr[:7], scr[7:14]
    src_pos_v, f_off_v, row0_v = scr[14:]
    wid = lax.axis_index("s") * NC + lax.axis_index("c")
    pltpu.sync_copy(seq_tab.at[pl.ds(0, 1)], row0_v)
    lanes = lax.iota(jnp.int32, 16)

    for j in range(ROWS // 16):
        pos = lanes + (j * 16)
        f = pos % (F + 1)
        smp = pos // (F + 1)
        src_pos_v[pl.ds(j * 16, 16)] = jnp.minimum(smp * F + f, SPID - 1)
        f_off_v[pl.ds(j * 16, 16)] = jnp.where(f < F, f * V, -1)

    def stage1(c, bufs):
        ids_v, idx_v, sqid_v, rows_v, seq_v, gsem, osem = bufs
        s_base = wid * SPW + c * S
        pltpu.sync_copy(sp_ids.at[pl.ds(s_base * F, SPID)], ids_v)
        pltpu.sync_copy(sq_ids.at[pl.ds(s_base * L, SEQN)], sqid_v)
        for j in range(ROWS // 16):
            sp = src_pos_v[pl.ds(j * 16, 16)]
            offv = f_off_v[pl.ds(j * 16, 16)]
            idv = plsc.load_gather(ids_v, [sp])
            idx_v[pl.ds(j * 16, 16)] = jnp.where(offv >= 0, idv + offv, 0)

        @pl.when(c >= 2)
        def _():
            old = (wid * SPW + (c - 2) * S) * (F + 1)
            pltpu.make_async_copy(rows_v, out.at[pl.ds(old, ROWS)], osem).wait()

        for off in range(0, ROWS, GSUB):
            n = min(GSUB, ROWS - off)
            pltpu.async_copy(
                tables.at[idx_v.at[pl.ds(off, n)]], rows_v.at[pl.ds(off, n)], gsem)
        for off in range(0, SEQN, GSUB):
            n = min(GSUB, SEQN - off)
            pltpu.async_copy(
                seq_tab.at[sqid_v.at[pl.ds(off, n)]], seq_v.at[pl.ds(off, n)], gsem)

    def stage2(c, bufs):
        ids_v, idx_v, sqid_v, rows_v, seq_v, gsem, osem = bufs
        s_base = wid * SPW + c * S
        for off in range(0, ROWS, GSUB):
            n = min(GSUB, ROWS - off)
            pltpu.make_async_copy(
                tables.at[idx_v.at[pl.ds(off, n)]], rows_v.at[pl.ds(off, n)], gsem).wait()
        for off in range(0, SEQN, GSUB):
            n = min(GSUB, SEQN - off)
            pltpu.make_async_copy(
                seq_tab.at[sqid_v.at[pl.ds(off, n)]], seq_v.at[pl.ds(off, n)], gsem).wait()

        r0a = row0_v[0, pl.ds(0, 16)]
        r0b = row0_v[0, pl.ds(16, 16)]

        def sample_body(s, carry2):
            ob = s * L
            i0 = sqid_v[pl.ds(ob, 16)]
            i1 = sqid_v[pl.ds(ob + 16, 16)]
            i2 = sqid_v[pl.ds(ob + 32, 16)]
            i3 = sqid_v[pl.ds(ob + 34, 16)]
            cnt = (plsc.all_reduce_population_count(i0 != 0)
                   + plsc.all_reduce_population_count(i1 != 0)
                   + plsc.all_reduce_population_count(i2 != 0)
                   + plsc.all_reduce_population_count(
                       jnp.logical_and(i3 != 0, lanes >= 14)))
            acc0 = jnp.zeros((16,), jnp.float32)
            acc1 = jnp.zeros((16,), jnp.float32)
            for l in range(L):
                acc0 = acc0 + seq_v[ob + l, pl.ds(0, 16)]
                acc1 = acc1 + seq_v[ob + l, pl.ds(16, 16)]
            cf = cnt.astype(jnp.float32)
            nz = 50.0 - cf
            denom = jnp.maximum(cf, 1.0)
            p0 = jnp.where(cnt > 0, (acc0 - nz * r0a) / denom, 0.0)
            p1 = jnp.where(cnt > 0, (acc1 - nz * r0b) / denom, 0.0)
            orow = s * (F + 1) + F
            rows_v[orow, pl.ds(0, 16)] = p0
            rows_v[orow, pl.ds(16, 16)] = p1
            return carry2

        lax.fori_loop(0, S, sample_body, 0)
        pltpu.async_copy(rows_v, out.at[pl.ds(s_base * (F + 1), ROWS)], osem)

    stage1(0, b0)

    def body(k, carry):
        c0 = 2 * k
        stage1(c0 + 1, b1)
        stage2(c0, b0)

        @pl.when(c0 + 2 < NCHUNK)
        def _():
            stage1(c0 + 2, b0)

        stage2(c0 + 1, b1)
        return carry

    lax.fori_loop(0, NCHUNK // 2, body, 0)

    last0 = (wid * SPW + (NCHUNK - 2) * S) * (F + 1)
    last1 = (wid * SPW + (NCHUNK - 1) * S) * (F + 1)
    pltpu.make_async_copy(b0[3], out.at[pl.ds(last0, ROWS)], b0[6]).wait()
    pltpu.make_async_copy(b1[3], out.at[pl.ds(last1, ROWS)], b1[6]).wait()


def kernel(sparse_ids, seq_ids, sparse_tables, seq_table):
    sp = sparse_ids.astype(jnp.int32).reshape(B * F)
    sq = seq_ids.astype(jnp.int32).reshape(B * L)
    tabs = sparse_tables.reshape(F * V, D)
    out = _sc_embed(sp, sq, tabs, seq_table)
    return out.reshape(B, F + 1, D)
